# baseline (speedup 1.0000x reference)
import jax
import jax.numpy as jnp
from jax.experimental import pallas as pl

F32 = jnp.float32
BF16 = jnp.bfloat16

D = 128
KEIG = 256
NCLS = 9

BN = 512
BI = 1024
BJ = 1024


def _dot(a, b):
    return jax.lax.dot_general(a, b, (((1,), (0,)), ((), ())),
                               preferred_element_type=F32)


def _dot_tn(a, b):
    return jax.lax.dot_general(a, b, (((0,), (0,)), ((), ())),
                               preferred_element_type=F32)


def _wspec(r, c):
    return pl.BlockSpec((r, c), lambda *_: (0, 0))


def _rspec(bn, c):
    return pl.BlockSpec((bn, c), lambda i: (i, 0))


def _row_a3_body(x_ref, e_ref, wid_ref, wmsg_ref, t_ref, z_ref, xw_ref):
    i = pl.program_id(0)
    x = x_ref[...]

    @pl.when(i == 0)
    def _():
        t_ref[...] = jnp.zeros_like(t_ref)

    t_ref[...] += _dot_tn(e_ref[...].astype(BF16), x.astype(BF16))
    z_ref[...] = _dot(x, wid_ref[...])
    xw_ref[...] = _dot(x, wmsg_ref[...])


def _row_a3(x, e, wid, wmsg):
    n, ke = e.shape
    return pl.pallas_call(
        _row_a3_body,
        grid=(n // BN,),
        in_specs=[_rspec(BN, D), _rspec(BN, ke), _wspec(D, D), _wspec(D, D)],
        out_specs=(_wspec(ke, D), _rspec(BN, D), _rspec(BN, D)),
        out_shape=(jax.ShapeDtypeStruct((ke, D), F32),
                   jax.ShapeDtypeStruct((n, D), F32),
                   jax.ShapeDtypeStruct((n, D), F32)),
    )(x, e, wid, wmsg)


def _row_a2_body(x_ref, e_ref, wid_ref, t_ref, z_ref):
    i = pl.program_id(0)
    x = x_ref[...]

    @pl.when(i == 0)
    def _():
        t_ref[...] = jnp.zeros_like(t_ref)

    t_ref[...] += _dot_tn(e_ref[...].astype(BF16), x.astype(BF16))
    z_ref[...] = _dot(x, wid_ref[...])


def _row_a2(x, e, wid):
    n, ke = e.shape
    return pl.pallas_call(
        _row_a2_body,
        grid=(n // BN,),
        in_specs=[_rspec(BN, D), _rspec(BN, ke), _wspec(D, D)],
        out_specs=(_wspec(ke, D), _rspec(BN, D)),
        out_shape=(jax.ShapeDtypeStruct((ke, D), F32),
                   jax.ShapeDtypeStruct((n, D), F32)),
    )(x, e, wid)


def _mini_mm_body(x_ref, w_ref, o_ref):
    o_ref[...] = _dot(x_ref[...], w_ref[...])


def _mini_mm(x, w):
    n = x.shape[0]
    return pl.pallas_call(
        _mini_mm_body,
        grid=(n // BN,),
        in_specs=[_rspec(BN, D), _wspec(D, D)],
        out_specs=_rspec(BN, D),
        out_shape=jax.ShapeDtypeStruct((n, D), F32),
    )(x, w)


def _s_b_body(t0_ref, t1_ref, e0_ref, ed_ref, eu_ref,
              w01_ref, w02_ref, wd0_ref, wd1_ref, wu0_ref, wu1_ref,
              s0_ref, s1_ref):
    e0 = e0_ref[...]
    t0 = t0_ref[...]
    s0_ref[...] = _dot(e0 * t0, w01_ref[...]) + _dot(e0 * e0 * t0, w02_ref[...])
    ed = ed_ref[...]
    eu = eu_ref[...]
    td = t1_ref[0:KEIG, :]
    tu = t1_ref[KEIG:2 * KEIG, :]
    s1_ref[0:KEIG, :] = _dot(ed * td, wd0_ref[...]) + _dot(ed * ed * td, wd1_ref[...])
    s1_ref[KEIG:2 * KEIG, :] = _dot(eu * tu, wu0_ref[...]) + _dot(eu * eu * tu, wu1_ref[...])


def _s_b(t0, t1, e0, ed, eu, w01, w02, wd0, wd1, wu0, wu1):
    return pl.pallas_call(
        _s_b_body,
        in_specs=[_wspec(KEIG, D), _wspec(2 * KEIG, D), _wspec(KEIG, 1),
                  _wspec(KEIG, 1), _wspec(KEIG, 1),
                  _wspec(D, D), _wspec(D, D), _wspec(D, D), _wspec(D, D),
                  _wspec(D, D), _wspec(D, D)],
        out_specs=(_wspec(KEIG, D), _wspec(2 * KEIG, D)),
        out_shape=(jax.ShapeDtypeStruct((KEIG, D), F32),
                   jax.ShapeDtypeStruct((2 * KEIG, D), F32)),
    )(t0, t1, e0, ed, eu, w01, w02, wd0, wd1, wu0, wu1)


def _s_b0_body(t0_ref, e0_ref, w01_ref, w02_ref, s0_ref):
    e0 = e0_ref[...]
    t0 = t0_ref[...]
    s0_ref[...] = _dot(e0 * t0, w01_ref[...]) + _dot(e0 * e0 * t0, w02_ref[...])


def _s_b0(t0, e0, w01, w02):
    return pl.pallas_call(
        _s_b0_body,
        in_specs=[_wspec(KEIG, D), _wspec(KEIG, 1), _wspec(D, D), _wspec(D, D)],
        out_specs=_wspec(KEIG, D),
        out_shape=jax.ShapeDtypeStruct((KEIG, D), F32),
    )(t0, e0, w01, w02)


def _c_dual_body(inc_ref, u_ref, v_ref, y0_ref, y1_ref):
    i = pl.program_id(0)
    j = pl.program_id(1)
    inc = inc_ref[...].astype(BF16)
    a = _dot(inc, u_ref[...].astype(BF16))
    b = _dot_tn(inc, v_ref[...].astype(BF16))

    @pl.when(j == 0)
    def _():
        y0_ref[...] = a

    @pl.when(j > 0)
    def _():
        y0_ref[...] += a

    @pl.when(i == 0)
    def _():
        y1_ref[pl.ds(j * BJ, BJ), :] = b

    @pl.when(i > 0)
    def _():
        y1_ref[pl.ds(j * BJ, BJ), :] += b


def _c_dual(inc, u, v):
    n0, n1 = inc.shape
    return pl.pallas_call(
        _c_dual_body,
        grid=(n0 // BI, n1 // BJ),
        in_specs=[pl.BlockSpec((BI, BJ), lambda i, j: (i, j)),
                  pl.BlockSpec((BJ, D), lambda i, j: (j, 0)),
                  pl.BlockSpec((BI, D), lambda i, j: (i, 0))],
        out_specs=(pl.BlockSpec((BI, D), lambda i, j: (i, 0)),
                   pl.BlockSpec((n1, D), lambda i, j: (0, 0))),
        out_shape=(jax.ShapeDtypeStruct((n0, D), F32),
                   jax.ShapeDtypeStruct((n1, D), F32)),
    )(inc, u, v)


def _c_mm_body(inc_ref, u_ref, y_ref):
    j = pl.program_id(1)
    a = _dot(inc_ref[...].astype(BF16), u_ref[...].astype(BF16))

    @pl.when(j == 0)
    def _():
        y_ref[...] = a

    @pl.when(j > 0)
    def _():
        y_ref[...] += a


def _c_mm(inc, u):
    n0, n1 = inc.shape
    return pl.pallas_call(
        _c_mm_body,
        grid=(n0 // BI, n1 // BJ),
        in_specs=[pl.BlockSpec((BI, BJ), lambda i, j: (i, j)),
                  pl.BlockSpec((BJ, D), lambda i, j: (j, 0))],
        out_specs=pl.BlockSpec((BI, D), lambda i, j: (i, 0)),
        out_shape=jax.ShapeDtypeStruct((n0, D), F32),
    )(inc, u)


def _d_act1_body(z_ref, ya_ref, e_ref, s_ref, o_ref):
    acc = z_ref[...] + ya_ref[...] + _dot(e_ref[...].astype(BF16),
                                          s_ref[...].astype(BF16))
    o_ref[...] = jax.nn.sigmoid(acc)


def _d_act1(z, ya, e, s):
    n, ke = e.shape
    return pl.pallas_call(
        _d_act1_body,
        grid=(n // BN,),
        in_specs=[_rspec(BN, D), _rspec(BN, D), _rspec(BN, ke), _wspec(ke, D)],
        out_specs=_rspec(BN, D),
        out_shape=jax.ShapeDtypeStruct((n, D), F32),
    )(z, ya, e, s)


def _d_act2_body(z_ref, ya_ref, yb_ref, e_ref, s_ref, o_ref):
    acc = (z_ref[...] + ya_ref[...] + yb_ref[...]
           + _dot(e_ref[...].astype(BF16), s_ref[...].astype(BF16)))
    o_ref[...] = jax.nn.sigmoid(acc)


def _d_act2(z, ya, yb, e, s):
    n, ke = e.shape
    return pl.pallas_call(
        _d_act2_body,
        grid=(n // BN,),
        in_specs=[_rspec(BN, D), _rspec(BN, D), _rspec(BN, D), _rspec(BN, ke),
                  _wspec(ke, D)],
        out_specs=_rspec(BN, D),
        out_shape=jax.ShapeDtypeStruct((n, D), F32),
    )(z, ya, yb, e, s)


def _d_head_body(z_ref, ya_ref, e_ref, s_ref, wout_ref, bout_ref, o_ref):
    acc = z_ref[...] + ya_ref[...] + _dot(e_ref[...].astype(BF16),
                                          s_ref[...].astype(BF16))
    x0f = jax.nn.sigmoid(acc)
    o_ref[...] = _dot(x0f, wout_ref[...]) + bout_ref[...]


def _d_head(z, ya, e, s, wout, bout2):
    n, ke = e.shape
    return pl.pallas_call(
        _d_head_body,
        grid=(n // BN,),
        in_specs=[_rspec(BN, D), _rspec(BN, D), _rspec(BN, ke), _wspec(ke, D),
                  _wspec(D, NCLS), _wspec(1, NCLS)],
        out_specs=_rspec(BN, NCLS),
        out_shape=jax.ShapeDtypeStruct((n, NCLS), F32),
    )(z, ya, e, s, wout, bout2)


def kernel(x_0, x_1, x_2, evals_0, evecs_0, evals_d1, evecs_d1, evals_u1,
           evecs_u1, evals_d2, evecs_d2, evals_u2, evecs_u2, incidence_1,
           incidence_2, W0, W10, W1id, W1d, W1u, W01, W21, W2id, W2d, W2u,
           W12, Wout, bout):
    e0 = evecs_0
    e1 = jnp.concatenate([evecs_d1, evecs_u1], axis=1)
    ev0 = evals_0.reshape(KEIG, 1)
    evd1 = evals_d1.reshape(KEIG, 1)
    evu1 = evals_u1.reshape(KEIG, 1)

    t0, z0, xw01 = _row_a3(x_0, e0, W0[0, 0], W01[0])
    t1, z1, xw10 = _row_a3(x_1, e1, W1id[0], W10[0])
    xw21 = _mini_mm(x_2, W21[0])
    s0, s1 = _s_b(t0, t1, ev0, evd1, evu1, W0[0, 1], W0[0, 2],
                  W1d[0, 0], W1d[0, 1], W1u[0, 0], W1u[0, 1])
    y0m, y1a = _c_dual(incidence_1, xw10, xw01)
    y1b = _c_mm(incidence_2, xw21)
    x0n = _d_act1(z0, y0m, e0, s0)
    x1n = _d_act2(z1, y1a, y1b, e1, s1)

    t0b, z0b = _row_a2(x0n, e0, W0[1, 0])
    xw10b = _mini_mm(x1n, W10[1])
    s0b = _s_b0(t0b, ev0, W0[1, 1], W0[1, 2])
    y0mb = _c_mm(incidence_1, xw10b)
    logits = _d_head(z0b, y0mb, e0, s0b, Wout, bout.reshape(1, NCLS))
    return logits


# speedup vs baseline: 1.4474x; 1.4474x over previous
import jax
import jax.numpy as jnp
from jax.experimental import pallas as pl
from jax.experimental.pallas import tpu as pltpu

F32 = jnp.float32
BF16 = jnp.bfloat16

D = 128
KEIG = 256
NCLS = 9
N0, N1, N2 = 2048, 6144, 4096

BI1, BJ1 = 1024, 1024
BI2, BJ2 = 1024, 1024
NBI1, NBJ1 = N0 // BI1, N1 // BJ1
NBI2, NBJ2 = N1 // BI2, N2 // BJ2

P1_LO = 1
P1_HI = P1_LO + NBI1 * NBJ1 - 1
P2_LO = P1_HI + 1
P2_HI = P2_LO + NBI2 * NBJ2 - 1
P3 = P2_HI + 1
P4_LO = P3 + 1
P4_HI = P4_LO + NBI1 * NBJ1 - 1
P5 = P4_HI + 1
NSTEPS = P5 + 1


def _dot(a, b):
    return jax.lax.dot_general(a, b, (((1,), (0,)), ((), ())),
                               preferred_element_type=F32)


def _dot_tn(a, b):
    return jax.lax.dot_general(a, b, (((0,), (0,)), ((), ())),
                               preferred_element_type=F32)


def _net_body(x0, x1, x2, e0, e1, ev0, evd1, evu1,
              w000, w001, w002, w010, w011, w012,
              w01_0, w10_0, w10_1, w21_0, w1id0,
              wd00, wd01, wu00, wu01, wout, bout2,
              inc1, inc2,
              out,
              xw01, xw10, xw21, y0m, y1acc, x0n, s0s, s1s, y0mb):
    s = pl.program_id(0)

    @pl.when(s == 0)
    def _prep():
        x0v = x0[...]
        x1v = x1[...]
        xw01[...] = _dot(x0v, w01_0[...])
        xw10[...] = _dot(x1v, w10_0[...])
        xw21[...] = _dot(x2[...], w21_0[...])
        t0 = _dot_tn(e0[...].astype(BF16), x0v.astype(BF16))
        t1 = _dot_tn(e1[...].astype(BF16), x1v.astype(BF16))
        e0v = ev0[...]
        s0s[...] = _dot(e0v * t0, w001[...]) + _dot(e0v * e0v * t0, w002[...])
        ed = evd1[...]
        eu = evu1[...]
        td = t1[0:KEIG, :]
        tu = t1[KEIG:2 * KEIG, :]
        s1s[0:KEIG, :] = _dot(ed * td, wd00[...]) + _dot(ed * ed * td, wd01[...])
        s1s[KEIG:2 * KEIG, :] = _dot(eu * tu, wu00[...]) + _dot(eu * eu * tu, wu01[...])

    @pl.when((s >= P1_LO) & (s <= P1_HI))
    def _pass1():
        t = s - P1_LO
        i = t // NBJ1
        j = t % NBJ1
        inc = inc1[...].astype(BF16)
        u = xw10[pl.ds(j * BJ1, BJ1), :].astype(BF16)
        v = xw01[pl.ds(i * BI1, BI1), :].astype(BF16)
        a = _dot(inc, u)
        b = _dot_tn(inc, v)

        @pl.when(j == 0)
        def _():
            y0m[pl.ds(i * BI1, BI1), :] = a

        @pl.when(j > 0)
        def _():
            y0m[pl.ds(i * BI1, BI1), :] += a

        @pl.when(i == 0)
        def _():
            y1acc[pl.ds(j * BJ1, BJ1), :] = b

        @pl.when(i > 0)
        def _():
            y1acc[pl.ds(j * BJ1, BJ1), :] += b

    @pl.when((s >= P2_LO) & (s <= P2_HI))
    def _pass2():
        t = s - P2_LO
        i = t // NBJ2
        j = t % NBJ2
        inc = inc2[...].astype(BF16)
        u = xw21[pl.ds(j * BJ2, BJ2), :].astype(BF16)
        y1acc[pl.ds(i * BI2, BI2), :] += _dot(inc, u)

    @pl.when(s == P3)
    def _combine0():
        x0v = x0[...]
        x1v = x1[...]
        y0 = (_dot(x0v, w000[...]) + y0m[...]
              + _dot(e0[...].astype(BF16), s0s[...].astype(BF16)))
        x0nv = jax.nn.sigmoid(y0)
        x0n[...] = x0nv
        y1 = (_dot(x1v, w1id0[...]) + y1acc[...]
              + _dot(e1[...].astype(BF16), s1s[...].astype(BF16)))
        x1nv = jax.nn.sigmoid(y1)
        xw10[...] = _dot(x1nv, w10_1[...])
        t0b = _dot_tn(e0[...].astype(BF16), x0nv.astype(BF16))
        e0v = ev0[...]
        s0s[...] = _dot(e0v * t0b, w011[...]) + _dot(e0v * e0v * t0b, w012[...])

    @pl.when((s >= P4_LO) & (s <= P4_HI))
    def _pass4():
        t = s - P4_LO
        i = t // NBJ1
        j = t % NBJ1
        inc = inc1[...].astype(BF16)
        u = xw10[pl.ds(j * BJ1, BJ1), :].astype(BF16)
        a = _dot(inc, u)

        @pl.when(j == 0)
        def _():
            y0mb[pl.ds(i * BI1, BI1), :] = a

        @pl.when(j > 0)
        def _():
            y0mb[pl.ds(i * BI1, BI1), :] += a

    @pl.when(s == P5)
    def _head():
        y0 = (_dot(x0n[...], w010[...]) + y0mb[...]
              + _dot(e0[...].astype(BF16), s0s[...].astype(BF16)))
        x0f = jax.nn.sigmoid(y0)
        out[...] = _dot(x0f, wout[...]) + bout2[...]


def _inc1_map(s):
    sa = jnp.clip(s, P1_LO, P1_HI) - P1_LO
    sb = jnp.clip(s, P4_LO, P4_HI) - P4_LO
    t = jnp.where(s >= P4_LO, sb, sa)
    return (t // NBJ1, t % NBJ1)


def _inc2_map(s):
    t = jnp.clip(s, P2_LO, P2_HI) - P2_LO
    return (t // NBJ2, t % NBJ2)


def _whole(r, c):
    return pl.BlockSpec((r, c), lambda s: (0, 0))


def kernel(x_0, x_1, x_2, evals_0, evecs_0, evals_d1, evecs_d1, evals_u1,
           evecs_u1, evals_d2, evecs_d2, evals_u2, evecs_u2, incidence_1,
           incidence_2, W0, W10, W1id, W1d, W1u, W01, W21, W2id, W2d, W2u,
           W12, Wout, bout):
    e1 = jnp.concatenate([evecs_d1, evecs_u1], axis=1)
    ev0 = evals_0.reshape(KEIG, 1)
    evd1 = evals_d1.reshape(KEIG, 1)
    evu1 = evals_u1.reshape(KEIG, 1)

    in_specs = [
        _whole(N0, D), _whole(N1, D), _whole(N2, D),
        _whole(N0, KEIG), _whole(N1, 2 * KEIG),
        _whole(KEIG, 1), _whole(KEIG, 1), _whole(KEIG, 1),
        _whole(D, D), _whole(D, D), _whole(D, D),
        _whole(D, D), _whole(D, D), _whole(D, D),
        _whole(D, D), _whole(D, D), _whole(D, D),
        _whole(D, D), _whole(D, D),
        _whole(D, D), _whole(D, D), _whole(D, D), _whole(D, D),
        _whole(D, NCLS), _whole(1, NCLS),
        pl.BlockSpec((BI1, BJ1), _inc1_map),
        pl.BlockSpec((BI2, BJ2), _inc2_map),
    ]
    scratch = [
        pltpu.VMEM((N0, D), F32),
        pltpu.VMEM((N1, D), F32),
        pltpu.VMEM((N2, D), F32),
        pltpu.VMEM((N0, D), F32),
        pltpu.VMEM((N1, D), F32),
        pltpu.VMEM((N0, D), F32),
        pltpu.VMEM((KEIG, D), F32),
        pltpu.VMEM((2 * KEIG, D), F32),
        pltpu.VMEM((N0, D), F32),
    ]
    return pl.pallas_call(
        _net_body,
        grid=(NSTEPS,),
        in_specs=in_specs,
        out_specs=_whole(N0, NCLS),
        out_shape=jax.ShapeDtypeStruct((N0, NCLS), F32),
        scratch_shapes=scratch,
    )(x_0, x_1, x_2, evecs_0, e1, ev0, evd1, evu1,
      W0[0, 0], W0[0, 1], W0[0, 2], W0[1, 0], W0[1, 1], W0[1, 2],
      W01[0], W10[0], W10[1], W21[0], W1id[0],
      W1d[0, 0], W1d[0, 1], W1u[0, 0], W1u[0, 1], Wout,
      bout.reshape(1, NCLS), incidence_1, incidence_2)


# speedup vs baseline: 1.6606x; 1.1473x over previous
import jax
import jax.numpy as jnp
from jax.experimental import pallas as pl
from jax.experimental.pallas import tpu as pltpu

F32 = jnp.float32
BF16 = jnp.bfloat16

D = 128
KEIG = 256
NCLS = 9
N0, N1, N2 = 2048, 6144, 4096

BP1 = 256
BP2 = 256
NP1 = N0 // BP1
NP2 = N1 // BP2

P1_LO = 1
P1_HI = P1_LO + NP1 - 1
P2_LO = P1_HI + 1
P2_HI = P2_LO + NP2 - 1
P3 = P2_HI + 1
P4_LO = P3 + 1
P4_HI = P4_LO + NP1 - 1
P5 = P4_HI + 1
NSTEPS = P5 + 1


def _dot(a, b):
    return jax.lax.dot_general(a, b, (((1,), (0,)), ((), ())),
                               preferred_element_type=F32)


def _dot_tn(a, b):
    return jax.lax.dot_general(a, b, (((0,), (0,)), ((), ())),
                               preferred_element_type=F32)


def _net_body(x0, x1, x2, e0, ed1, eu1, ev0, evd1, evu1,
              w000, w001, w002, w010, w011, w012,
              w01_0, w10_0, w10_1, w21_0, w1id0,
              wd00, wd01, wu00, wu01, wout, bout2,
              inc1, inc2,
              out,
              xw01, xw10, xw21, y0m, y1acc, x0n, s0s, s1s, y0mb):
    s = pl.program_id(0)

    @pl.when(s == 0)
    def _prep():
        x0v = x0[...]
        x1v = x1[...]
        xw01[...] = _dot(x0v, w01_0[...]).astype(BF16)
        xw10[...] = _dot(x1v, w10_0[...]).astype(BF16)
        xw21[...] = _dot(x2[...], w21_0[...]).astype(BF16)
        x1b = x1v.astype(BF16)
        t0 = _dot_tn(e0[...].astype(BF16), x0v.astype(BF16))
        td = _dot_tn(ed1[...].astype(BF16), x1b)
        tu = _dot_tn(eu1[...].astype(BF16), x1b)
        e0v = ev0[...]
        s0s[...] = (_dot(e0v * t0, w001[...])
                    + _dot(e0v * e0v * t0, w002[...])).astype(BF16)
        ed = evd1[...]
        eu = evu1[...]
        s1s[0:KEIG, :] = (_dot(ed * td, wd00[...])
                          + _dot(ed * ed * td, wd01[...])).astype(BF16)
        s1s[KEIG:2 * KEIG, :] = (_dot(eu * tu, wu00[...])
                                 + _dot(eu * eu * tu, wu01[...])).astype(BF16)

    @pl.when((s >= P1_LO) & (s <= P1_HI))
    def _pass1():
        i = s - P1_LO
        inc = inc1[...].astype(BF16)
        y0m[pl.ds(i * BP1, BP1), :] = _dot(inc, xw10[...])
        b = _dot_tn(inc, xw01[pl.ds(i * BP1, BP1), :])

        @pl.when(i == 0)
        def _():
            y1acc[...] = b

        @pl.when(i > 0)
        def _():
            y1acc[...] += b

    @pl.when((s >= P2_LO) & (s <= P2_HI))
    def _pass2():
        i = s - P2_LO
        inc = inc2[...].astype(BF16)
        y1acc[pl.ds(i * BP2, BP2), :] += _dot(inc, xw21[...])

    @pl.when(s == P3)
    def _combine0():
        x0v = x0[...]
        x1v = x1[...]
        y0 = (_dot(x0v, w000[...]) + y0m[...]
              + _dot(e0[...].astype(BF16), s0s[...]))
        x0nv = jax.nn.sigmoid(y0)
        x0n[...] = x0nv
        y1 = (_dot(x1v, w1id0[...]) + y1acc[...]
              + _dot(ed1[...].astype(BF16), s1s[0:KEIG, :])
              + _dot(eu1[...].astype(BF16), s1s[KEIG:2 * KEIG, :]))
        x1nv = jax.nn.sigmoid(y1)
        xw10[...] = _dot(x1nv, w10_1[...]).astype(BF16)
        t0b = _dot_tn(e0[...].astype(BF16), x0nv.astype(BF16))
        e0v = ev0[...]
        s0s[...] = (_dot(e0v * t0b, w011[...])
                    + _dot(e0v * e0v * t0b, w012[...])).astype(BF16)

    @pl.when((s >= P4_LO) & (s <= P4_HI))
    def _pass4():
        i = s - P4_LO
        inc = inc1[...].astype(BF16)
        y0mb[pl.ds(i * BP1, BP1), :] = _dot(inc, xw10[...])

    @pl.when(s == P5)
    def _head():
        y0 = (_dot(x0n[...], w010[...]) + y0mb[...]
              + _dot(e0[...].astype(BF16), s0s[...]))
        x0f = jax.nn.sigmoid(y0)
        out[...] = _dot(x0f, wout[...]) + bout2[...]


def _inc1_map(s):
    sa = jnp.clip(s, P1_LO, P1_HI) - P1_LO
    sb = jnp.clip(s, P4_LO, P4_HI) - P4_LO
    return (jnp.where(s >= P4_LO, sb, sa), 0)


def _inc2_map(s):
    return (jnp.clip(s, P2_LO, P2_HI) - P2_LO, 0)


def _whole(r, c):
    return pl.BlockSpec((r, c), lambda s: (0, 0))


def kernel(x_0, x_1, x_2, evals_0, evecs_0, evals_d1, evecs_d1, evals_u1,
           evecs_u1, evals_d2, evecs_d2, evals_u2, evecs_u2, incidence_1,
           incidence_2, W0, W10, W1id, W1d, W1u, W01, W21, W2id, W2d, W2u,
           W12, Wout, bout):
    ev0 = evals_0.reshape(KEIG, 1)
    evd1 = evals_d1.reshape(KEIG, 1)
    evu1 = evals_u1.reshape(KEIG, 1)

    in_specs = [
        _whole(N0, D), _whole(N1, D), _whole(N2, D),
        _whole(N0, KEIG), _whole(N1, KEIG), _whole(N1, KEIG),
        _whole(KEIG, 1), _whole(KEIG, 1), _whole(KEIG, 1),
        _whole(D, D), _whole(D, D), _whole(D, D),
        _whole(D, D), _whole(D, D), _whole(D, D),
        _whole(D, D), _whole(D, D), _whole(D, D),
        _whole(D, D), _whole(D, D),
        _whole(D, D), _whole(D, D), _whole(D, D), _whole(D, D),
        _whole(D, NCLS), _whole(1, NCLS),
        pl.BlockSpec((BP1, N1), _inc1_map),
        pl.BlockSpec((BP2, N2), _inc2_map),
    ]
    scratch = [
        pltpu.VMEM((N0, D), BF16),
        pltpu.VMEM((N1, D), BF16),
        pltpu.VMEM((N2, D), BF16),
        pltpu.VMEM((N0, D), F32),
        pltpu.VMEM((N1, D), F32),
        pltpu.VMEM((N0, D), F32),
        pltpu.VMEM((KEIG, D), BF16),
        pltpu.VMEM((2 * KEIG, D), BF16),
        pltpu.VMEM((N0, D), F32),
    ]
    return pl.pallas_call(
        _net_body,
        grid=(NSTEPS,),
        in_specs=in_specs,
        out_specs=_whole(N0, NCLS),
        out_shape=jax.ShapeDtypeStruct((N0, NCLS), F32),
        scratch_shapes=scratch,
    )(x_0, x_1, x_2, evecs_0, evecs_d1, evecs_u1, ev0, evd1, evu1,
      W0[0, 0], W0[0, 1], W0[0, 2], W0[1, 0], W0[1, 1], W0[1, 2],
      W01[0], W10[0], W10[1], W21[0], W1id[0],
      W1d[0, 0], W1d[0, 1], W1u[0, 0], W1u[0, 1], Wout,
      bout.reshape(1, NCLS), incidence_1, incidence_2)


# speedup vs baseline: 1.8197x; 1.0958x over previous
import jax
import jax.numpy as jnp
from jax.experimental import pallas as pl
from jax.experimental.pallas import tpu as pltpu

F32 = jnp.float32
BF16 = jnp.bfloat16

D = 128
KEIG = 256
NCLS = 9
N0, N1, N2 = 2048, 6144, 4096

BP1 = 256
BP2 = 256
NP1 = N0 // BP1
NP2 = N1 // BP2

P1_LO = 1
P1_HI = P1_LO + NP1 - 1
P2_LO = P1_HI + 1
P2_HI = P2_LO + NP2 - 1
P3 = P2_HI + 1
P4_LO = P3 + 1
P4_HI = P4_LO + NP1 - 1
P5 = P4_HI + 1
NSTEPS = P5 + 1


def _dot(a, b):
    return jax.lax.dot_general(a, b, (((1,), (0,)), ((), ())),
                               preferred_element_type=F32)


def _dot_tn(a, b):
    return jax.lax.dot_general(a, b, (((0,), (0,)), ((), ())),
                               preferred_element_type=F32)


def _net_body(x0, x1, x2, e0, ed1, eu1, evs,
              w0, w10, w1id, w1d, w1u, w01, w21, wout, bout,
              inc1, inc2,
              out,
              xw01, xw10, xw21, y0m, y1acc, x0n, s0s, s1s, y0mb):
    s = pl.program_id(0)

    @pl.when(s == 0)
    def _prep():
        x0v = x0[...]
        x1v = x1[...]
        xw01[...] = _dot(x0v, w01[0]).astype(BF16)
        xw10[...] = _dot(x1v, w10[0]).astype(BF16)
        xw21[...] = _dot(x2[...], w21[0]).astype(BF16)
        x1b = x1v.astype(BF16)
        t0 = _dot_tn(e0[...].astype(BF16), x0v.astype(BF16))
        td = _dot_tn(ed1[...].astype(BF16), x1b)
        tu = _dot_tn(eu1[...].astype(BF16), x1b)
        e0v = evs[0]
        s0s[...] = (_dot(e0v * t0, w0[0, 1])
                    + _dot(e0v * e0v * t0, w0[0, 2])).astype(BF16)
        ed = evs[1]
        eu = evs[2]
        s1s[0:KEIG, :] = (_dot(ed * td, w1d[0, 0])
                          + _dot(ed * ed * td, w1d[0, 1])).astype(BF16)
        s1s[KEIG:2 * KEIG, :] = (_dot(eu * tu, w1u[0, 0])
                                 + _dot(eu * eu * tu, w1u[0, 1])).astype(BF16)

    @pl.when((s >= P1_LO) & (s <= P1_HI))
    def _pass1():
        i = s - P1_LO
        inc = inc1[...].astype(BF16)
        y0m[pl.ds(i * BP1, BP1), :] = _dot(inc, xw10[...])
        b = _dot_tn(inc, xw01[pl.ds(i * BP1, BP1), :])

        @pl.when(i == 0)
        def _():
            y1acc[...] = b

        @pl.when(i > 0)
        def _():
            y1acc[...] += b

    @pl.when((s >= P2_LO) & (s <= P2_HI))
    def _pass2():
        i = s - P2_LO
        inc = inc2[...].astype(BF16)
        y1acc[pl.ds(i * BP2, BP2), :] += _dot(inc, xw21[...])

    @pl.when(s == P3)
    def _combine0():
        x0v = x0[...]
        x1v = x1[...]
        y0 = (_dot(x0v, w0[0, 0]) + y0m[...]
              + _dot(e0[...].astype(BF16), s0s[...]))
        x0nv = jax.nn.sigmoid(y0)
        x0n[...] = x0nv
        y1 = (_dot(x1v, w1id[0]) + y1acc[...]
              + _dot(ed1[...].astype(BF16), s1s[0:KEIG, :])
              + _dot(eu1[...].astype(BF16), s1s[KEIG:2 * KEIG, :]))
        x1nv = jax.nn.sigmoid(y1)
        xw10[...] = _dot(x1nv, w10[1]).astype(BF16)
        t0b = _dot_tn(e0[...].astype(BF16), x0nv.astype(BF16))
        e0v = evs[0]
        s0s[...] = (_dot(e0v * t0b, w0[1, 1])
                    + _dot(e0v * e0v * t0b, w0[1, 2])).astype(BF16)

    @pl.when((s >= P4_LO) & (s <= P4_HI))
    def _pass4():
        i = s - P4_LO
        inc = inc1[...].astype(BF16)
        y0mb[pl.ds(i * BP1, BP1), :] = _dot(inc, xw10[...])

    @pl.when(s == P5)
    def _head():
        y0 = (_dot(x0n[...], w0[1, 0]) + y0mb[...]
              + _dot(e0[...].astype(BF16), s0s[...]))
        x0f = jax.nn.sigmoid(y0)
        out[...] = _dot(x0f, wout[...]) + bout[...]


def _inc1_map(s):
    sa = jnp.clip(s, P1_LO, P1_HI) - P1_LO
    sb = jnp.clip(s, P4_LO, P4_HI) - P4_LO
    return (jnp.where(s >= P4_LO, sb, sa), 0)


def _inc2_map(s):
    return (jnp.clip(s, P2_LO, P2_HI) - P2_LO, 0)


def _whole(*shape):
    return pl.BlockSpec(shape, lambda s: (0,) * len(shape))


def kernel(x_0, x_1, x_2, evals_0, evecs_0, evals_d1, evecs_d1, evals_u1,
           evecs_u1, evals_d2, evecs_d2, evals_u2, evecs_u2, incidence_1,
           incidence_2, W0, W10, W1id, W1d, W1u, W01, W21, W2id, W2d, W2u,
           W12, Wout, bout):
    evs = jnp.stack([evals_0, evals_d1, evals_u1], axis=0).reshape(3, KEIG, 1)

    in_specs = [
        _whole(N0, D), _whole(N1, D), _whole(N2, D),
        _whole(N0, KEIG), _whole(N1, KEIG), _whole(N1, KEIG),
        _whole(3, KEIG, 1),
        _whole(2, 3, D, D),
        _whole(2, D, D), _whole(2, D, D),
        _whole(2, 2, D, D), _whole(2, 2, D, D),
        _whole(2, D, D), _whole(2, D, D),
        _whole(D, NCLS), pl.BlockSpec((NCLS,), lambda s: (0,)),
        pl.BlockSpec((BP1, N1), _inc1_map),
        pl.BlockSpec((BP2, N2), _inc2_map),
    ]
    scratch = [
        pltpu.VMEM((N0, D), BF16),
        pltpu.VMEM((N1, D), BF16),
        pltpu.VMEM((N2, D), BF16),
        pltpu.VMEM((N0, D), F32),
        pltpu.VMEM((N1, D), F32),
        pltpu.VMEM((N0, D), F32),
        pltpu.VMEM((KEIG, D), BF16),
        pltpu.VMEM((2 * KEIG, D), BF16),
        pltpu.VMEM((N0, D), F32),
    ]
    return pl.pallas_call(
        _net_body,
        grid=(NSTEPS,),
        in_specs=in_specs,
        out_specs=_whole(N0, NCLS),
        out_shape=jax.ShapeDtypeStruct((N0, NCLS), F32),
        scratch_shapes=scratch,
        compiler_params=pltpu.CompilerParams(
            vmem_limit_bytes=63 * 1024 * 1024),
    )(x_0, x_1, x_2, evecs_0, evecs_d1, evecs_u1, evs,
      W0, W10, W1id, W1d, W1u, W01, W21, Wout, bout,
      incidence_1, incidence_2)
